# baseline (speedup 1.0000x reference)
import jax
import jax.numpy as jnp
from jax.experimental import pallas as pl
from jax.experimental.pallas import tpu as pltpu

_TOTAL_COMPONENTS = 128
_TOTAL_NODES = 50000
_TOTAL_EDGES = 800000

_FEAT_COPY_CHUNKS = 10
_FEAT_FILL_CHUNKS = 5
_EDGE_COPY_CHUNKS = 2
_EDGE_FILL_CHUNKS = 5


def kernel(node_features, edge_index, node_sizes, edge_sizes):
    num_nodes, d = node_features.shape
    num_edges = edge_index.shape[1]
    num_components = node_sizes.shape[0]
    pad_nodes = _TOTAL_NODES - num_nodes
    pad_edges = _TOTAL_EDGES - num_edges

    feat_copy_rows = num_nodes // _FEAT_COPY_CHUNKS
    feat_fill_rows = pad_nodes // _FEAT_FILL_CHUNKS
    edge_copy_cols = num_edges // _EDGE_COPY_CHUNKS
    edge_fill_cols = pad_edges // _EDGE_FILL_CHUNKS
    n_dma = (_FEAT_COPY_CHUNKS + _FEAT_FILL_CHUNKS
             + _EDGE_COPY_CHUNKS + _EDGE_FILL_CHUNKS)

    def body(nf_ref, ei_ref, ns_ref, es_ref,
             pf_ref, pei_ref, pns_ref, pes_ref, mask_ref,
             zeros_scr, efill_scr, sems):
        zeros_scr[...] = jnp.zeros_like(zeros_scr)
        efill_scr[...] = jnp.full_like(efill_scr, num_nodes)

        copies = []
        s = 0
        for i in range(_FEAT_COPY_CHUNKS):
            r0 = i * feat_copy_rows
            copies.append(pltpu.make_async_copy(
                nf_ref.at[pl.ds(r0, feat_copy_rows)],
                pf_ref.at[pl.ds(r0, feat_copy_rows)],
                sems.at[s]))
            s += 1
        for i in range(_FEAT_FILL_CHUNKS):
            r0 = num_nodes + i * feat_fill_rows
            copies.append(pltpu.make_async_copy(
                zeros_scr,
                pf_ref.at[pl.ds(r0, feat_fill_rows)],
                sems.at[s]))
            s += 1
        for i in range(_EDGE_COPY_CHUNKS):
            c0 = i * edge_copy_cols
            copies.append(pltpu.make_async_copy(
                ei_ref.at[:, pl.ds(c0, edge_copy_cols)],
                pei_ref.at[:, pl.ds(c0, edge_copy_cols)],
                sems.at[s]))
            s += 1
        for i in range(_EDGE_FILL_CHUNKS):
            c0 = num_edges + i * edge_fill_cols
            copies.append(pltpu.make_async_copy(
                efill_scr,
                pei_ref.at[:, pl.ds(c0, edge_fill_cols)],
                sems.at[s]))
            s += 1
        for c in copies:
            c.start()

        lane = jax.lax.broadcasted_iota(jnp.int32, (1, _TOTAL_COMPONENTS), 1)
        zpad = jnp.zeros((1, _TOTAL_COMPONENTS - num_components), jnp.int32)
        ns_row = jnp.concatenate([ns_ref[...], zpad], axis=1)
        es_row = jnp.concatenate([es_ref[...], zpad], axis=1)
        pns_ref[...] = jnp.where(lane == num_components, pad_nodes, ns_row)
        pes_ref[...] = jnp.where(lane == num_components, pad_edges, es_row)
        mask_ref[...] = lane < num_components

        for c in copies:
            c.wait()

    out = pl.pallas_call(
        body,
        out_shape=[
            jax.ShapeDtypeStruct((_TOTAL_NODES, d), node_features.dtype),
            jax.ShapeDtypeStruct((2, _TOTAL_EDGES), edge_index.dtype),
            jax.ShapeDtypeStruct((1, _TOTAL_COMPONENTS), node_sizes.dtype),
            jax.ShapeDtypeStruct((1, _TOTAL_COMPONENTS), edge_sizes.dtype),
            jax.ShapeDtypeStruct((1, _TOTAL_COMPONENTS), jnp.bool_),
        ],
        in_specs=[
            pl.BlockSpec(memory_space=pl.ANY),
            pl.BlockSpec(memory_space=pl.ANY),
            pl.BlockSpec(memory_space=pltpu.MemorySpace.VMEM),
            pl.BlockSpec(memory_space=pltpu.MemorySpace.VMEM),
        ],
        out_specs=[
            pl.BlockSpec(memory_space=pl.ANY),
            pl.BlockSpec(memory_space=pl.ANY),
            pl.BlockSpec(memory_space=pltpu.MemorySpace.VMEM),
            pl.BlockSpec(memory_space=pltpu.MemorySpace.VMEM),
            pl.BlockSpec(memory_space=pltpu.MemorySpace.VMEM),
        ],
        scratch_shapes=[
            pltpu.VMEM((feat_fill_rows, d), node_features.dtype),
            pltpu.VMEM((2, edge_fill_cols), edge_index.dtype),
            pltpu.SemaphoreType.DMA((n_dma,)),
        ],
    )(node_features, edge_index,
      node_sizes.reshape(1, num_components),
      edge_sizes.reshape(1, num_components))

    padded_features, padded_edge_index, pns, pes, mask = out
    return (
        padded_features,
        padded_edge_index,
        pns.reshape(_TOTAL_COMPONENTS),
        pes.reshape(_TOTAL_COMPONENTS),
        mask.reshape(_TOTAL_COMPONENTS),
    )


# speedup vs baseline: 13.2093x; 13.2093x over previous
import jax
import jax.numpy as jnp
from jax.experimental import pallas as pl
from jax.experimental.pallas import tpu as pltpu

_TOTAL_COMPONENTS = 128
_TOTAL_NODES = 50000
_TOTAL_EDGES = 800000

_GRID = 32
_FB = 1600
_EB = 200
_COPY_BLOCKS = 25


def kernel(node_features, edge_index, node_sizes, edge_sizes):
    num_nodes, d = node_features.shape
    num_edges = edge_index.shape[1]
    num_components = node_sizes.shape[0]
    pad_nodes = _TOTAL_NODES - num_nodes
    pad_edges = _TOTAL_EDGES - num_edges

    e_rows_in = num_edges // 128
    e_rows_out = _TOTAL_EDGES // 128

    def body(nf_ref, ei_ref, ns_ref, es_ref,
             pf_ref, pei_ref, pns_ref, pes_ref, mask_ref):
        i = pl.program_id(0)
        is_copy = i < _COPY_BLOCKS
        pf_ref[...] = jnp.where(is_copy, nf_ref[...], 0.0)
        pei_ref[...] = jnp.where(is_copy, ei_ref[...], num_nodes)

        @pl.when(i == 0)
        def _():
            lane = jax.lax.broadcasted_iota(
                jnp.int32, (1, _TOTAL_COMPONENTS), 1)
            zpad = jnp.zeros((1, _TOTAL_COMPONENTS - num_components),
                             jnp.int32)
            ns_row = jnp.concatenate([ns_ref[...], zpad], axis=1)
            es_row = jnp.concatenate([es_ref[...], zpad], axis=1)
            pns_ref[...] = jnp.where(lane == num_components, pad_nodes,
                                     ns_row)
            pes_ref[...] = jnp.where(lane == num_components, pad_edges,
                                     es_row)
            mask_ref[...] = lane < num_components

    clamp = _COPY_BLOCKS - 1

    out = pl.pallas_call(
        body,
        grid=(_GRID,),
        out_shape=[
            jax.ShapeDtypeStruct((_TOTAL_NODES, d), node_features.dtype),
            jax.ShapeDtypeStruct((2, e_rows_out, 128), edge_index.dtype),
            jax.ShapeDtypeStruct((1, _TOTAL_COMPONENTS), node_sizes.dtype),
            jax.ShapeDtypeStruct((1, _TOTAL_COMPONENTS), edge_sizes.dtype),
            jax.ShapeDtypeStruct((1, _TOTAL_COMPONENTS), jnp.bool_),
        ],
        in_specs=[
            pl.BlockSpec((_FB, d), lambda i: (jnp.minimum(i, clamp), 0)),
            pl.BlockSpec((2, _EB, 128),
                         lambda i: (0, jnp.minimum(i, clamp), 0)),
            pl.BlockSpec((1, num_components), lambda i: (0, 0)),
            pl.BlockSpec((1, num_components), lambda i: (0, 0)),
        ],
        out_specs=[
            pl.BlockSpec((_FB, d), lambda i: (i, 0)),
            pl.BlockSpec((2, _EB, 128), lambda i: (0, i, 0)),
            pl.BlockSpec((1, _TOTAL_COMPONENTS), lambda i: (0, 0)),
            pl.BlockSpec((1, _TOTAL_COMPONENTS), lambda i: (0, 0)),
            pl.BlockSpec((1, _TOTAL_COMPONENTS), lambda i: (0, 0)),
        ],
    )(node_features,
      edge_index.reshape(2, e_rows_in, 128),
      node_sizes.reshape(1, num_components),
      edge_sizes.reshape(1, num_components))

    padded_features, pei_3d, pns, pes, mask = out
    return (
        padded_features,
        pei_3d.reshape(2, _TOTAL_EDGES),
        pns.reshape(_TOTAL_COMPONENTS),
        pes.reshape(_TOTAL_COMPONENTS),
        mask.reshape(_TOTAL_COMPONENTS),
    )


# speedup vs baseline: 24.0808x; 1.8230x over previous
import jax
import jax.numpy as jnp
from jax.experimental import pallas as pl
from jax.experimental.pallas import tpu as pltpu

_TOTAL_COMPONENTS = 128
_TOTAL_NODES = 50000
_TOTAL_EDGES = 800000

_GRID = 32
_FB = 1600
_ELB = 25600
_COPY_BLOCKS = 25


def kernel(node_features, edge_index, node_sizes, edge_sizes):
    num_nodes, d = node_features.shape
    num_edges = edge_index.shape[1]
    num_components = node_sizes.shape[0]
    pad_nodes = _TOTAL_NODES - num_nodes
    pad_edges = _TOTAL_EDGES - num_edges

    def body(nf_ref, ei_ref, pf_ref, pei_ref):
        i = pl.program_id(0)
        is_copy = i < _COPY_BLOCKS
        pf_ref[...] = jnp.where(is_copy, nf_ref[...], 0.0)
        pei_ref[...] = jnp.where(is_copy, ei_ref[...], num_nodes)

    clamp = _COPY_BLOCKS - 1

    padded_features, padded_edge_index = pl.pallas_call(
        body,
        grid=(_GRID,),
        out_shape=[
            jax.ShapeDtypeStruct((_TOTAL_NODES, d), node_features.dtype),
            jax.ShapeDtypeStruct((2, _TOTAL_EDGES), edge_index.dtype),
        ],
        in_specs=[
            pl.BlockSpec((_FB, d), lambda i: (jnp.minimum(i, clamp), 0)),
            pl.BlockSpec((2, _ELB), lambda i: (0, jnp.minimum(i, clamp))),
        ],
        out_specs=[
            pl.BlockSpec((_FB, d), lambda i: (i, 0)),
            pl.BlockSpec((2, _ELB), lambda i: (0, i)),
        ],
    )(node_features, edge_index)

    padded_node_sizes = (
        jnp.zeros((_TOTAL_COMPONENTS,), dtype=node_sizes.dtype)
        .at[:num_components].set(node_sizes)
        .at[num_components].set(jnp.asarray(pad_nodes, node_sizes.dtype)))
    padded_edge_sizes = (
        jnp.zeros((_TOTAL_COMPONENTS,), dtype=edge_sizes.dtype)
        .at[:num_components].set(edge_sizes)
        .at[num_components].set(jnp.asarray(pad_edges, edge_sizes.dtype)))
    component_mask = jnp.arange(_TOTAL_COMPONENTS) < num_components

    return (
        padded_features,
        padded_edge_index,
        padded_node_sizes,
        padded_edge_sizes,
        component_mask,
    )


# speedup vs baseline: 27.0291x; 1.1224x over previous
import jax
import jax.numpy as jnp
from jax.experimental import pallas as pl
from jax.experimental.pallas import tpu as pltpu

_TOTAL_COMPONENTS = 128
_TOTAL_NODES = 50000
_TOTAL_EDGES = 800000

_GRID = 25
_FB = 2000
_ELB = 32000
_COPY_BLOCKS = 20


def kernel(node_features, edge_index, node_sizes, edge_sizes):
    num_nodes, d = node_features.shape
    num_edges = edge_index.shape[1]
    num_components = node_sizes.shape[0]
    pad_nodes = _TOTAL_NODES - num_nodes
    pad_edges = _TOTAL_EDGES - num_edges

    def body(nf_ref, ei_ref, pf_ref, pei_ref):
        i = pl.program_id(0)
        is_copy = i < _COPY_BLOCKS
        pf_ref[...] = jnp.where(is_copy, nf_ref[...], 0.0)
        pei_ref[...] = jnp.where(is_copy, ei_ref[...], num_nodes)

    clamp = _COPY_BLOCKS - 1

    padded_features, padded_edge_index = pl.pallas_call(
        body,
        grid=(_GRID,),
        out_shape=[
            jax.ShapeDtypeStruct((_TOTAL_NODES, d), node_features.dtype),
            jax.ShapeDtypeStruct((2, _TOTAL_EDGES), edge_index.dtype),
        ],
        in_specs=[
            pl.BlockSpec((_FB, d), lambda i: (jnp.minimum(i, clamp), 0)),
            pl.BlockSpec((2, _ELB), lambda i: (0, jnp.minimum(i, clamp))),
        ],
        out_specs=[
            pl.BlockSpec((_FB, d), lambda i: (i, 0)),
            pl.BlockSpec((2, _ELB), lambda i: (0, i)),
        ],
    )(node_features, edge_index)

    padded_node_sizes = (
        jnp.zeros((_TOTAL_COMPONENTS,), dtype=node_sizes.dtype)
        .at[:num_components].set(node_sizes)
        .at[num_components].set(jnp.asarray(pad_nodes, node_sizes.dtype)))
    padded_edge_sizes = (
        jnp.zeros((_TOTAL_COMPONENTS,), dtype=edge_sizes.dtype)
        .at[:num_components].set(edge_sizes)
        .at[num_components].set(jnp.asarray(pad_edges, edge_sizes.dtype)))
    component_mask = jnp.arange(_TOTAL_COMPONENTS) < num_components

    return (
        padded_features,
        padded_edge_index,
        padded_node_sizes,
        padded_edge_sizes,
        component_mask,
    )


# speedup vs baseline: 34.2963x; 1.2689x over previous
import jax
import jax.numpy as jnp
from jax.experimental import pallas as pl
from jax.experimental.pallas import tpu as pltpu

_TOTAL_COMPONENTS = 128
_TOTAL_NODES = 50000
_TOTAL_EDGES = 800000

_GRID = 10
_FB = 5000
_ELB = 80000
_COPY_BLOCKS = 8


def kernel(node_features, edge_index, node_sizes, edge_sizes):
    num_nodes, d = node_features.shape
    num_edges = edge_index.shape[1]
    num_components = node_sizes.shape[0]
    pad_nodes = _TOTAL_NODES - num_nodes
    pad_edges = _TOTAL_EDGES - num_edges

    def body(nf_ref, ei_ref, pf_ref, pei_ref):
        i = pl.program_id(0)
        is_copy = i < _COPY_BLOCKS
        pf_ref[...] = jnp.where(is_copy, nf_ref[...], 0.0)
        pei_ref[...] = jnp.where(is_copy, ei_ref[...], num_nodes)

    clamp = _COPY_BLOCKS - 1

    padded_features, padded_edge_index = pl.pallas_call(
        body,
        grid=(_GRID,),
        out_shape=[
            jax.ShapeDtypeStruct((_TOTAL_NODES, d), node_features.dtype),
            jax.ShapeDtypeStruct((2, _TOTAL_EDGES), edge_index.dtype),
        ],
        in_specs=[
            pl.BlockSpec((_FB, d), lambda i: (jnp.minimum(i, clamp), 0)),
            pl.BlockSpec((2, _ELB), lambda i: (0, jnp.minimum(i, clamp))),
        ],
        out_specs=[
            pl.BlockSpec((_FB, d), lambda i: (i, 0)),
            pl.BlockSpec((2, _ELB), lambda i: (0, i)),
        ],
    )(node_features, edge_index)

    padded_node_sizes = (
        jnp.zeros((_TOTAL_COMPONENTS,), dtype=node_sizes.dtype)
        .at[:num_components].set(node_sizes)
        .at[num_components].set(jnp.asarray(pad_nodes, node_sizes.dtype)))
    padded_edge_sizes = (
        jnp.zeros((_TOTAL_COMPONENTS,), dtype=edge_sizes.dtype)
        .at[:num_components].set(edge_sizes)
        .at[num_components].set(jnp.asarray(pad_edges, edge_sizes.dtype)))
    component_mask = jnp.arange(_TOTAL_COMPONENTS) < num_components

    return (
        padded_features,
        padded_edge_index,
        padded_node_sizes,
        padded_edge_sizes,
        component_mask,
    )


# speedup vs baseline: 36.2947x; 1.0583x over previous
import jax
import jax.numpy as jnp
from jax.experimental import pallas as pl
from jax.experimental.pallas import tpu as pltpu

_TOTAL_COMPONENTS = 128
_TOTAL_NODES = 50000
_TOTAL_EDGES = 800000

_GRID = 5
_FB = 10000
_ELB = 160000
_COPY_BLOCKS = 4


def kernel(node_features, edge_index, node_sizes, edge_sizes):
    num_nodes, d = node_features.shape
    num_edges = edge_index.shape[1]
    num_components = node_sizes.shape[0]
    pad_nodes = _TOTAL_NODES - num_nodes
    pad_edges = _TOTAL_EDGES - num_edges

    def body(nf_ref, ei_ref, pf_ref, pei_ref):
        i = pl.program_id(0)
        is_copy = i < _COPY_BLOCKS
        pf_ref[...] = jnp.where(is_copy, nf_ref[...], 0.0)
        pei_ref[...] = jnp.where(is_copy, ei_ref[...], num_nodes)

    clamp = _COPY_BLOCKS - 1

    padded_features, padded_edge_index = pl.pallas_call(
        body,
        grid=(_GRID,),
        out_shape=[
            jax.ShapeDtypeStruct((_TOTAL_NODES, d), node_features.dtype),
            jax.ShapeDtypeStruct((2, _TOTAL_EDGES), edge_index.dtype),
        ],
        in_specs=[
            pl.BlockSpec((_FB, d), lambda i: (jnp.minimum(i, clamp), 0)),
            pl.BlockSpec((2, _ELB), lambda i: (0, jnp.minimum(i, clamp))),
        ],
        out_specs=[
            pl.BlockSpec((_FB, d), lambda i: (i, 0)),
            pl.BlockSpec((2, _ELB), lambda i: (0, i)),
        ],
    )(node_features, edge_index)

    padded_node_sizes = (
        jnp.zeros((_TOTAL_COMPONENTS,), dtype=node_sizes.dtype)
        .at[:num_components].set(node_sizes)
        .at[num_components].set(jnp.asarray(pad_nodes, node_sizes.dtype)))
    padded_edge_sizes = (
        jnp.zeros((_TOTAL_COMPONENTS,), dtype=edge_sizes.dtype)
        .at[:num_components].set(edge_sizes)
        .at[num_components].set(jnp.asarray(pad_edges, edge_sizes.dtype)))
    component_mask = jnp.arange(_TOTAL_COMPONENTS) < num_components

    return (
        padded_features,
        padded_edge_index,
        padded_node_sizes,
        padded_edge_sizes,
        component_mask,
    )


# speedup vs baseline: 39.0237x; 1.0752x over previous
import jax
import jax.numpy as jnp
from jax.experimental import pallas as pl
from jax.experimental.pallas import tpu as pltpu

_TOTAL_COMPONENTS = 128
_TOTAL_NODES = 50000
_TOTAL_EDGES = 800000

_GRID = 3
_FB = 20000
_ELB = 320000
_COPY_BLOCKS = 2


def kernel(node_features, edge_index, node_sizes, edge_sizes):
    num_nodes, d = node_features.shape
    num_edges = edge_index.shape[1]
    num_components = node_sizes.shape[0]
    pad_nodes = _TOTAL_NODES - num_nodes
    pad_edges = _TOTAL_EDGES - num_edges

    def body(nf_ref, ei_ref, pf_ref, pei_ref):
        i = pl.program_id(0)
        is_copy = i < _COPY_BLOCKS
        pf_ref[...] = jnp.where(is_copy, nf_ref[...], 0.0)
        pei_ref[...] = jnp.where(is_copy, ei_ref[...], num_nodes)

    clamp = _COPY_BLOCKS - 1

    padded_features, padded_edge_index = pl.pallas_call(
        body,
        grid=(_GRID,),
        out_shape=[
            jax.ShapeDtypeStruct((_TOTAL_NODES, d), node_features.dtype),
            jax.ShapeDtypeStruct((2, _TOTAL_EDGES), edge_index.dtype),
        ],
        in_specs=[
            pl.BlockSpec((_FB, d), lambda i: (jnp.minimum(i, clamp), 0)),
            pl.BlockSpec((2, _ELB), lambda i: (0, jnp.minimum(i, clamp))),
        ],
        out_specs=[
            pl.BlockSpec((_FB, d), lambda i: (i, 0)),
            pl.BlockSpec((2, _ELB), lambda i: (0, i)),
        ],
    )(node_features, edge_index)

    padded_node_sizes = (
        jnp.zeros((_TOTAL_COMPONENTS,), dtype=node_sizes.dtype)
        .at[:num_components].set(node_sizes)
        .at[num_components].set(jnp.asarray(pad_nodes, node_sizes.dtype)))
    padded_edge_sizes = (
        jnp.zeros((_TOTAL_COMPONENTS,), dtype=edge_sizes.dtype)
        .at[:num_components].set(edge_sizes)
        .at[num_components].set(jnp.asarray(pad_edges, edge_sizes.dtype)))
    component_mask = jnp.arange(_TOTAL_COMPONENTS) < num_components

    return (
        padded_features,
        padded_edge_index,
        padded_node_sizes,
        padded_edge_sizes,
        component_mask,
    )
